# Initial kernel scaffold; baseline (speedup 1.0000x reference)
import jax
import jax.numpy as jnp
from jax.experimental import pallas as pl


def kernel(unary_tensor, binary_tensor, W_emb, b_emb, Wroot0, Wrel0, b0, Wroot1, Wrel1, b1, W_pol, b_pol, W_base, b_base):
    raise NotImplementedError("write your pallas kernel here")


# speedup vs baseline: 10032.2022x; 10032.2022x over previous
import jax
import jax.numpy as jnp
from jax.experimental import pallas as pl


def _gnn_kernel(un_ref, adj_ref, wemb_ref, bemb_ref,
                wroot0_ref, wrel0_ref, b0_ref,
                wroot1_ref, wrel1_ref, b1_ref,
                wpol_ref, bpol_ref, wbase_ref, bbase_ref,
                logits_ref, base_ref, act_ref):
    f32 = jnp.float32
    R = adj_ref.shape[1]
    A = wpol_ref.shape[1]

    x = jax.lax.dot_general(un_ref[0], wemb_ref[...],
                            (((1,), (0,)), ((), ())),
                            preferred_element_type=f32) + bemb_ref[...]

    adjh = []
    for r in range(R):
        a = adj_ref[0, r].astype(f32)
        deg = jnp.sum(a, axis=0, keepdims=True)
        adjh.append(a * (1.0 / jnp.maximum(deg, 1.0)))

    def rgcn(x, wroot, wrel, bias):
        agg = None
        for r in range(R):
            h = jax.lax.dot_general(x, wrel[r], (((1,), (0,)), ((), ())),
                                    preferred_element_type=f32)
            t = jax.lax.dot_general(adjh[r], h, (((0,), (0,)), ((), ())),
                                    preferred_element_type=f32)
            agg = t if agg is None else agg + t
        root = jax.lax.dot_general(x, wroot, (((1,), (0,)), ((), ())),
                                   preferred_element_type=f32)
        return jax.nn.relu(root + bias + agg)

    x = rgcn(x, wroot0_ref[...], wrel0_ref[...], b0_ref[...])
    x = rgcn(x, wroot1_ref[...], wrel1_ref[...], b1_ref[...])

    pooled = jnp.max(x, axis=0, keepdims=True)
    logits = jax.lax.dot_general(pooled, wpol_ref[...], (((1,), (0,)), ((), ())),
                                 preferred_element_type=f32) + bpol_ref[...]
    base = jax.lax.dot_general(pooled, wbase_ref[...], (((1,), (0,)), ((), ())),
                               preferred_element_type=f32) + bbase_ref[...]

    logits_ref[0] = logits
    base_ref[0] = base
    m = jnp.max(logits, axis=1, keepdims=True)
    iota = jax.lax.broadcasted_iota(jnp.int32, logits.shape, 1)
    act_ref[0] = jnp.min(jnp.where(logits == m, iota, A), axis=1, keepdims=True)


def kernel(unary_tensor, binary_tensor, W_emb, b_emb, Wroot0, Wrel0, b0,
           Wroot1, Wrel1, b1, W_pol, b_pol, W_base, b_base):
    Tt, Bb, N, F = unary_tensor.shape
    R = binary_tensor.shape[-1]
    D = W_emb.shape[1]
    A = W_pol.shape[1]
    BT = Tt * Bb
    f32 = jnp.float32

    un = unary_tensor.reshape(BT, N, F).astype(f32)
    adj = binary_tensor.reshape(BT, N, N, R).astype(jnp.int8).transpose(0, 3, 1, 2)

    full = lambda *shape: pl.BlockSpec(shape, lambda b: (0,) * len(shape))
    in_specs = [
        pl.BlockSpec((1, N, F), lambda b: (b, 0, 0)),
        pl.BlockSpec((1, R, N, N), lambda b: (b, 0, 0, 0)),
        full(F, D), full(1, D),
        full(D, D), full(R, D, D), full(1, D),
        full(D, D), full(R, D, D), full(1, D),
        full(D, A), full(1, A), full(D, 1), full(1, 1),
    ]
    out_specs = [
        pl.BlockSpec((1, 1, A), lambda b: (b, 0, 0)),
        pl.BlockSpec((1, 1, 1), lambda b: (b, 0, 0)),
        pl.BlockSpec((1, 1, 1), lambda b: (b, 0, 0)),
    ]
    logits, base, act = pl.pallas_call(
        _gnn_kernel,
        grid=(BT,),
        in_specs=in_specs,
        out_specs=out_specs,
        out_shape=[
            jax.ShapeDtypeStruct((BT, 1, A), f32),
            jax.ShapeDtypeStruct((BT, 1, 1), f32),
            jax.ShapeDtypeStruct((BT, 1, 1), jnp.int32),
        ],
    )(un, adj, W_emb, b_emb.reshape(1, D),
      Wroot0, Wrel0, b0.reshape(1, D),
      Wroot1, Wrel1, b1.reshape(1, D),
      W_pol, b_pol.reshape(1, A), W_base, b_base.reshape(1, 1))

    return (logits.reshape(Tt, Bb, A),
            base.reshape(Tt, Bb),
            act.reshape(Tt, Bb))
